# baseline (speedup 1.0000x reference)
import functools

import jax
import jax.numpy as jnp
from jax import lax
from jax.experimental import pallas as pl
from jax.experimental.pallas import tpu as pltpu
from jax.experimental.pallas import tpu_sc as plsc

N_USER = 20000
N_ITEM = 30000
N_BUNDLE = 10000
DIM = 128
HW = 64
NC = 2
NS = 16
L = 16

F32 = jnp.float32
I32 = jnp.int32


def _rpad(n):
    return ((n + 2047) // 2048) * 2048


NP_USER = _rpad(N_USER)
NP_ITEM = _rpad(N_ITEM)
NP_BUNDLE = _rpad(N_BUNDLE)


def _accrows(n):
    return ((n + 64 + 127) // 128) * 128


def _epad(e):
    return ((e + 16383) // 16384) * 16384


def _mesh():
    return plsc.VectorSubcoreMesh(
        core_axis_name="c", subcore_axis_name="s", num_cores=NC,
        num_subcores=NS)


_PARAMS = dict(compiler_params=pltpu.CompilerParams(
    use_tc_tiling_on_sc=False, needs_layout_passes=False))


def _fill(ref, rows, cols, value):
    v = jnp.full((L,), value, F32)
    nc = cols // L

    def body(k, _):
        i = k // nc
        j = k % nc
        ref[i, pl.ds(j * L, L)] = v
        return 0

    lax.fori_loop(0, rows * nc, body, 0)


@functools.lru_cache(maxsize=None)
def _make_deg(e_pad, na_pad, nb_pad):
    blocks_pt = e_pad // 128 // NS
    nmax = max(na_pad, nb_pad)

    def body(ia_hbm, ib_hbm, ra_hbm, rb_hbm, acc, idxb, ones_v, zb, ssem):
        c = lax.axis_index("c")
        s = lax.axis_index("s")

        _fill(zb, 128, 16, 0.0)
        _fill(ones_v, 128, 16, 1.0)

        def side(idx_hbm, out_hbm, n_pad):
            nblk = n_pad // 128

            def zb_body(i, _):
                b = s + i * NS
                pltpu.sync_copy(zb, acc.at[pl.ds(b * 128, 128)])
                return 0

            lax.fori_loop(0, nblk // NS, zb_body, 0)
            plsc.subcore_barrier()

            pltpu.sync_copy(idx_hbm.at[pl.ds(s * blocks_pt, blocks_pt)],
                            idxb)

            W = 8

            def sc_body(k, _):
                pltpu.async_copy(ones_v, acc.at[idxb.at[k]], ssem, add=True)

                @pl.when(k >= W)
                def _():
                    pltpu.make_async_copy(ones_v, acc.at[idxb.at[0]],
                                          ssem).wait()
                return 0

            lax.fori_loop(0, blocks_pt, sc_body, 0)

            def dr_body(k, _):
                pltpu.make_async_copy(ones_v, acc.at[idxb.at[0]],
                                      ssem).wait()
                return 0

            lax.fori_loop(0, min(W, blocks_pt), dr_body, 0)
            plsc.subcore_barrier()

            def out_body(i, _):
                b = s + i * NS
                pltpu.sync_copy(acc.at[pl.ds(b * 128, 128)], ones_v)
                pltpu.sync_copy(ones_v, out_hbm.at[pl.ds(b * 128, 128)])
                return 0

            lax.fori_loop(0, nblk // NS, out_body, 0)

        @pl.when(c == 0)
        def _():
            side(ia_hbm, ra_hbm, na_pad)

        @pl.when(c == 1)
        def _():
            side(ib_hbm, rb_hbm, nb_pad)

    return pl.kernel(
        body,
        out_type=(jax.ShapeDtypeStruct((na_pad, 16), F32),
                  jax.ShapeDtypeStruct((nb_pad, 16), F32)),
        mesh=_mesh(),
        scratch_types=[
            pltpu.VMEM_SHARED((nmax, 16), F32),
            pltpu.VMEM((blocks_pt, 128), I32),
            pltpu.VMEM((128, 16), F32),
            pltpu.VMEM((128, 16), F32),
            pltpu.SemaphoreType.DMA,
        ],
        **_PARAMS,
    )


def _seg_eb(acc_rows):
    return 64 if acc_rows * HW > 1_500_000 else 128


@functools.lru_cache(maxsize=None)
def _make_seg(e_pad, nsrc_pad, ndst_pad, acc_rows, accw, extra_ws):
    EB = _seg_eb(acc_rows)
    IB = 4 if EB == 64 else 8
    GR = EB // L
    blocks_pt = e_pad // EB // NS
    chunks_pt = blocks_pt // IB
    n_extra = len(extra_ws)
    acc_blk = acc_rows // EB

    def body(*args):
        (table, gidx_hbm, sidx_hbm, deg_hbm), rest = args[:4], args[4:]
        extras = rest[:n_extra]
        rest = rest[n_extra:]
        out_hbm = rest[0]
        acc, ga, gb_, da, db_, rba, rbb, dgb = rest[1:9]
        isa, isb, gsa, gsb = rest[9:13]

        c = lax.axis_index("c")
        s = lax.axis_index("s")
        base = s * blocks_pt
        off = c * nsrc_pad

        _fill(rba, EB, HW, 0.0)

        def z_body(i, _):
            b = s + i * NS

            @pl.when(b < acc_blk)
            def _():
                pltpu.sync_copy(rba, acc.at[pl.ds(b * EB, EB)])
            return 0

        lax.fori_loop(0, (acc_blk + NS - 1) // NS, z_body, 0)
        plsc.subcore_barrier()

        pltpu.sync_copy(
            gidx_hbm.at[pl.ds(base, IB)], ga)
        pltpu.sync_copy(
            sidx_hbm.at[pl.ds(base, IB)], da)

        if chunks_pt > 1:
            pltpu.async_copy(
                gidx_hbm.at[pl.ds(base + IB, IB)], gb_, isb)
            pltpu.async_copy(
                sidx_hbm.at[pl.ds(base + IB, IB)], db_, isb)

        def chunk(ci, gcur, dcur, gnxt, dnxt, my_isem, nxt_isem):
            def p_body(k, _):
                i = k // GR
                j = k % GR
                gcur[i, pl.ds(j * L, L)] = \
                    gcur[i, pl.ds(j * L, L)] + off
                return 0

            lax.fori_loop(0, IB * GR, p_body, 0)

            pltpu.async_copy(table.at[gcur.at[0]], rba, gsa)
            for k in range(IB):
                cur, csem = (rba, gsa) if k % 2 == 0 else (rbb, gsb)
                nxt, nsem = (rbb, gsb) if k % 2 == 0 else (rba, gsa)
                if k + 1 < IB:
                    pltpu.async_copy(table.at[gcur.at[k + 1]], nxt,
                                     nsem)
                pltpu.make_async_copy(table.at[gcur.at[0]], cur,
                                      csem).wait()
                pltpu.sync_copy(cur, acc.at[dcur.at[k]], add=True)

            @pl.when(ci + 2 < chunks_pt)
            def _():
                o = base + (ci + 2) * IB
                pltpu.async_copy(gidx_hbm.at[pl.ds(o, IB)], gcur,
                                 my_isem)
                pltpu.async_copy(sidx_hbm.at[pl.ds(o, IB)], dcur,
                                 my_isem)

            @pl.when(ci + 1 < chunks_pt)
            def _():
                pltpu.make_async_copy(
                    gidx_hbm.at[pl.ds(base, IB)], gnxt,
                    nxt_isem).wait()
                pltpu.make_async_copy(
                    sidx_hbm.at[pl.ds(base, IB)], dnxt,
                    nxt_isem).wait()

        def c_body(ci, _):
            @pl.when(ci % 2 == 0)
            def _():
                chunk(ci, ga, da, gb_, db_, isa, isb)

            @pl.when(ci % 2 == 1)
            def _():
                chunk(ci, gb_, db_, ga, da, isb, isa)
            return 0

        lax.fori_loop(0, chunks_pt, c_body, 0)
        plsc.subcore_barrier()

        out_off = c * ndst_pad

        def o_body(i, _):
            b = s + i * NS

            @pl.when(b < acc_blk)
            def _():
                r0 = b * EB
                pltpu.sync_copy(acc.at[pl.ds(r0, EB)], rba)
                pltpu.sync_copy(deg_hbm.at[pl.ds(r0, EB)], dgb)

                def row_body(ri, _):
                    dv = dgb[ri, pl.ds(0, L)]
                    sc = jnp.full((L,), accw, F32) \
                        / jnp.maximum(dv, 1.0)
                    for j in range(4):
                        rba[ri, pl.ds(j * L, L)] = \
                            rba[ri, pl.ds(j * L, L)] * sc
                    return 0

                lax.fori_loop(0, EB, row_body, 0)

                for t in range(n_extra):
                    pltpu.sync_copy(
                        extras[t].at[pl.ds(out_off + r0, EB)], rbb)

                    def e_body(k, _):
                        ri = k // 4
                        j = k % 4
                        rba[ri, pl.ds(j * L, L)] = \
                            rba[ri, pl.ds(j * L, L)] \
                            + rbb[ri, pl.ds(j * L, L)] * extra_ws[t]
                        return 0

                    lax.fori_loop(0, EB * 4, e_body, 0)

                pltpu.sync_copy(rba,
                                out_hbm.at[pl.ds(out_off + r0, EB)])
            return 0

        lax.fori_loop(0, (acc_blk + NS - 1) // NS, o_body, 0)

    def body2(*args):
        body(*args)

    scratch = [
        pltpu.VMEM_SHARED((acc_rows, HW), F32),
        pltpu.VMEM((IB, EB), I32),
        pltpu.VMEM((IB, EB), I32),
        pltpu.VMEM((IB, EB), I32),
        pltpu.VMEM((IB, EB), I32),
        pltpu.VMEM((EB, HW), F32),
        pltpu.VMEM((EB, HW), F32),
        pltpu.VMEM((EB, 16), F32),
        pltpu.SemaphoreType.DMA,
        pltpu.SemaphoreType.DMA,
        pltpu.SemaphoreType.DMA,
        pltpu.SemaphoreType.DMA,
    ]

    return pl.kernel(
        body2,
        out_type=jax.ShapeDtypeStruct((2 * ndst_pad, HW), F32),
        mesh=_mesh(),
        scratch_types=scratch,
        **_PARAMS,
    )


GB = 8


@functools.lru_cache(maxsize=None)
def _make_lit(n_seq, ntok_pad, nq_pad, t_real, t_pad, alpha, extra_ws,
              tok_full=False, full_out=False):
    n_extra = len(extra_ws)
    nblk = (n_seq + L - 1) // L
    ngrp = (nblk + GB - 1) // GB
    gseq = GB * L
    T = t_real
    NW = NC * NS
    nhb = 1 if tok_full else 2

    def body(*args):
        (tok, q_hbm, tidx_hbm), rest = args[:3], args[3:]
        extras = rest[:n_extra]
        rest = rest[n_extra:]
        out_hbm = rest[0]
        rest = rest[1:]
        if full_out:
            fout_hbm = rest[0]
            rest = rest[1:]
        ibuf = rest[0]
        qb = rest[1:3]
        eb = rest[3:3 + 2 * n_extra]
        rest = rest[3 + 2 * n_extra:]
        ob = rest[0:2]
        rest = rest[2:]
        if full_out:
            ofull = rest[0]
            rest = rest[1:]
        if tok_full:
            tba = rest[0:1]
            tbb = rest[1:2]
            rest = rest[2:]
            stA = stB = None
        else:
            stA = rest[0:2]
            stB = rest[2:4]
            tba = rest[4:6]
            tbb = rest[6:8]
            rest = rest[8:]
        gsa, gsb = rest[0], rest[1]

        c = lax.axis_index("c")
        s = lax.axis_index("s")
        wid = s * NC + c
        lane = lax.iota(I32, L)

        def tget(tq, t, u):
            if tok_full:
                return tq[0][t, pl.ds(u * L, L)]
            return tq[u // 4][t, pl.ds((u % 4) * L, L)]

        def do_group(g):
            s0 = g * gseq
            pltpu.sync_copy(tidx_hbm.at[pl.ds(s0 * t_pad, gseq * t_pad)],
                            ibuf)
            for h in range(2):
                r = h * nq_pad + s0
                pltpu.sync_copy(q_hbm.at[pl.ds(r, gseq)], qb[h])
                for t in range(n_extra):
                    pltpu.sync_copy(extras[t].at[pl.ds(r, gseq)],
                                    eb[t * 2 + h])

            def fire(sq, strips, bufs, sem):
                o = sq * t_pad
                if tok_full:
                    pltpu.async_copy(
                        tok.at[ibuf.at[pl.ds(o, T)]], bufs[0], sem)
                else:
                    nvr = t_pad // L
                    for h in range(2):
                        for j in range(nvr):
                            strips[h][pl.ds(j * L, L)] = \
                                ibuf[pl.ds(o + j * L, L)] + (h * ntok_pad)
                    for h in range(2):
                        pltpu.async_copy(
                            tok.at[strips[h].at[pl.ds(0, T)]], bufs[h],
                            sem)

            def drain(bufs, sem):
                if tok_full:
                    pltpu.make_async_copy(
                        tok.at[ibuf.at[pl.ds(0, T)]], bufs[0], sem).wait()
                else:
                    for h in range(2):
                        pltpu.make_async_copy(
                            tok.at[stA[h].at[pl.ds(0, T)]], bufs[h],
                            sem).wait()

            fire(0, stA, tba, gsa)

            def seq(sq, tq, cur_sem, nstrips, nbufs, nxt_sem):
                @pl.when(sq + 1 < gseq)
                def _():
                    fire(sq + 1, nstrips, nbufs, nxt_sem)

                drain(tq, cur_sem)

                qv = [qb[h][sq, pl.ds(j * L, L)]
                      for h in range(2) for j in range(4)]

                sc0 = jnp.full((L,), -1e30, F32)
                sc1 = jnp.full((L,), -1e30, F32)
                for t in range(T):
                    sv = qv[0] * tget(tq, t, 0)
                    for u in range(1, 8):
                        sv = sv + qv[u] * tget(tq, t, u)
                    sdot = jnp.sum(sv) * float(DIM ** -0.5)
                    msk = lane == (t % L)
                    if t < L:
                        sc0 = jnp.where(msk, sdot, sc0)
                    else:
                        sc1 = jnp.where(msk, sdot, sc1)

                if T > L:
                    m = jnp.maximum(jnp.max(sc0), jnp.max(sc1))
                    w0 = jnp.exp(sc0 - m)
                    w1 = jnp.exp(sc1 - m)
                    denom = jnp.sum(w0) + jnp.sum(w1)
                else:
                    m = jnp.max(sc0)
                    w0 = jnp.exp(sc0 - m)
                    w1 = w0
                    denom = jnp.sum(w0)

                rr = jnp.full((L,), 1.0 - alpha, F32) \
                    / (jnp.full((L,), 0.0, F32) + denom)

                accv = [jnp.zeros((L,), F32) for _ in range(8)]
                for t in range(T):
                    wsrc = w0 if t < L else w1
                    wt = wsrc.at[jnp.full((L,), t % L, I32)].get(
                        mode="promise_in_bounds")
                    for u in range(8):
                        accv[u] = accv[u] + wt * tget(tq, t, u)

                for u in range(8):
                    h, j = u // 4, u % 4
                    v = alpha * qv[u] + rr * accv[u]
                    for t in range(n_extra):
                        v = v + eb[t * 2 + h][sq, pl.ds(j * L, L)] \
                            * extra_ws[t]
                    ob[h][sq, pl.ds(j * L, L)] = v
                    if full_out:
                        ofull[sq, pl.ds(u * L, L)] = v

            def s_body(sq, _):
                @pl.when(sq % 2 == 0)
                def _():
                    seq(sq, tba, gsa, stB, tbb, gsb)

                @pl.when(sq % 2 == 1)
                def _():
                    seq(sq, tbb, gsb, stA, tba, gsa)
                return 0

            lax.fori_loop(0, gseq, s_body, 0)

            for h in range(2):
                pltpu.sync_copy(ob[h],
                                out_hbm.at[pl.ds(h * nq_pad + s0, gseq)])
            if full_out:
                pltpu.sync_copy(ofull, fout_hbm.at[pl.ds(s0, gseq)])

        def g_body(i, _):
            g = wid + i * NW

            @pl.when(g < ngrp)
            def _():
                do_group(g)
            return 0

        lax.fori_loop(0, (ngrp + NW - 1) // NW, g_body, 0)

    scratch = [pltpu.VMEM((gseq * t_pad,), I32)]
    scratch += [pltpu.VMEM((gseq, HW), F32) for _ in range(2)]
    scratch += [pltpu.VMEM((gseq, HW), F32) for _ in range(2 * n_extra)]
    scratch += [pltpu.VMEM((gseq, HW), F32) for _ in range(2)]
    if full_out:
        scratch += [pltpu.VMEM((gseq, DIM), F32)]
    if tok_full:
        scratch += [pltpu.VMEM((T, DIM), F32) for _ in range(2)]
    else:
        scratch += [pltpu.VMEM((t_pad,), I32) for _ in range(4)]
        scratch += [pltpu.VMEM((T, HW), F32) for _ in range(4)]
    scratch += [pltpu.SemaphoreType.DMA, pltpu.SemaphoreType.DMA]

    out_type = jax.ShapeDtypeStruct((2 * nq_pad, HW), F32)
    if full_out:
        out_type = (out_type, jax.ShapeDtypeStruct((nq_pad, DIM), F32))

    return pl.kernel(
        body,
        out_type=out_type,
        mesh=_mesh(),
        scratch_types=scratch,
        **_PARAMS,
    )


def _planarize(x, n_pad):
    n = x.shape[0]
    x = jnp.pad(x, ((0, n_pad - n), (0, 0)))
    return x.reshape(n_pad, 2, HW).transpose(1, 0, 2).reshape(
        2 * n_pad, HW)


def _unplanarize(x, n, n_pad):
    return x.reshape(2, n_pad, HW)[:, :n, :].transpose(1, 0, 2).reshape(
        n, DIM)


def _pad_edges(col, e_pad, pad_base, spread, eb):
    e = col.shape[0]
    if e < e_pad:
        pad = pad_base + (jnp.arange(e_pad - e, dtype=col.dtype) % spread)
        col = jnp.concatenate([col, pad])
    return col.reshape(e_pad // eb, eb)


def _prep_graph(edges, na, nb):
    e = edges.shape[1]
    ep = _epad(e)
    a = edges[0].astype(I32)
    b = edges[1].astype(I32)
    out = {"ep": ep}
    for eb in (64, 128):
        out[f"a_g{eb}"] = _pad_edges(a, ep, 0, min(na, 1024), eb)
        out[f"b_g{eb}"] = _pad_edges(b, ep, 0, min(nb, 1024), eb)
        out[f"a_s{eb}"] = _pad_edges(a, ep, na, 64, eb)
        out[f"b_s{eb}"] = _pad_edges(b, ep, nb, 64, eb)
    return out


def _pad_tokens(tidx, t_pad, s_pad):
    s, t = tidx.shape
    out = jnp.pad(tidx.astype(I32), ((0, s_pad - s), (0, t_pad - t)))
    return out.reshape(s_pad * t_pad)


def kernel(user_embedding, item_embedding, bundle_embedding, ui_edge_index,
           ub_edge_index, bi_edge_index, ub_token_idx, bi_token_idx):
    NU, NI, NB = NP_USER, NP_ITEM, NP_BUNDLE

    u0 = _planarize(user_embedding, NU)
    i0 = _planarize(item_embedding, NI)
    b0 = _planarize(bundle_embedding, NB)

    ui = _prep_graph(ui_edge_index, N_USER, N_ITEM)
    ub = _prep_graph(ub_edge_index, N_USER, N_BUNDLE)
    bi = _prep_graph(bi_edge_index, N_BUNDLE, N_ITEM)

    sp_u = ((N_USER + 127) // 128) * 128
    sp_b = ((N_BUNDLE + 127) // 128) * 128
    ub_tidx = _pad_tokens(ub_token_idx, 32, sp_u)
    bi_tidx = _pad_tokens(bi_token_idx, 16, sp_b)

    d_u_ui, d_i_ui = _make_deg(ui["ep"], NU, NI)(ui["a_s128"],
                                                 ui["b_s128"])
    d_u_ub, d_b_ub = _make_deg(ub["ep"], NU, NB)(ub["a_s128"],
                                                 ub["b_s128"])
    d_b_bi, d_i_bi = _make_deg(bi["ep"], NB, NI)(bi["a_s128"],
                                                 bi["b_s128"])

    def seg(table, gr, gcol, scol, deg, nsp, ndst, ndp, accw=1.0,
            extras=()):
        ws = tuple(float(w) for (_, w) in extras)
        arrs = tuple(a for (a, _) in extras)
        acc_rows = _accrows(ndst)
        eb = _seg_eb(acc_rows)
        g2 = gr[f"{gcol}_g{eb}"]
        s2 = gr[f"{scol}_s{eb}"]
        return _make_seg(gr["ep"], nsp, ndp, acc_rows, float(accw), ws)(
            table, g2, s2, deg, *arrs)

    u1 = seg(i0, ui, "b", "a", d_u_ui, NI, N_USER, NU)
    i1 = seg(u0, ui, "a", "b", d_i_ui, NU, N_ITEM, NI)
    u2 = seg(i1, ui, "b", "a", d_u_ui, NI, N_USER, NU)
    i2 = seg(u1, ui, "a", "b", d_i_ui, NU, N_ITEM, NI)
    u_cf = seg(i2, ui, "b", "a", d_u_ui, NI, N_USER, NU,
               accw=0.25, extras=((u0, 0.25), (u1, 0.25), (u2, 0.25)))
    i_cf = seg(u2, ui, "a", "b", d_i_ui, NU, N_ITEM, NI,
               accw=0.25, extras=((i0, 0.25), (i1, 0.25), (i2, 0.25)))

    u_f, i_f, b_f = u0, i0, b0
    for _ in range(3):
        u_b2u = seg(b_f, ub, "b", "a", d_u_ub, NB, N_USER, NU)
        b_u2b = seg(u_f, ub, "a", "b", d_b_ub, NU, N_BUNDLE, NB)
        b_i2b = seg(i_f, bi, "b", "a", d_b_bi, NI, N_BUNDLE, NB)
        i_new = seg(b_f, bi, "a", "b", d_i_bi, NB, N_ITEM, NI,
                    extras=((i_cf, 1.0),))
        b_new, b_full = _make_lit(N_BUNDLE, NI, NB, 5, 16, 0.5, (1.0,),
                                  full_out=True)(
            i_new, b_i2b, bi_tidx, b_u2b)
        u_new = _make_lit(N_USER, NB, NU, 30, 32, 0.5, (1.0,),
                          tok_full=True)(
            b_full, u_b2u, ub_tidx, u_cf)
        u_f, i_f, b_f = u_new, i_new, b_new

    return (_unplanarize(u_f, N_USER, NU),
            _unplanarize(i_f, N_ITEM, NI),
            _unplanarize(b_f, N_BUNDLE, NB))


# speedup vs baseline: 1.0245x; 1.0245x over previous
import functools

import jax
import jax.numpy as jnp
from jax import lax
from jax.experimental import pallas as pl
from jax.experimental.pallas import tpu as pltpu
from jax.experimental.pallas import tpu_sc as plsc

N_USER = 20000
N_ITEM = 30000
N_BUNDLE = 10000
DIM = 128
HW = 64
NC = 2
NS = 16
L = 16

F32 = jnp.float32
I32 = jnp.int32


def _rpad(n):
    return ((n + 2047) // 2048) * 2048


NP_USER = _rpad(N_USER)
NP_ITEM = _rpad(N_ITEM)
NP_BUNDLE = _rpad(N_BUNDLE)


def _accrows(n):
    return ((n + 64 + 127) // 128) * 128


def _epad(e):
    return ((e + 16383) // 16384) * 16384


def _mesh():
    return plsc.VectorSubcoreMesh(
        core_axis_name="c", subcore_axis_name="s", num_cores=NC,
        num_subcores=NS)


_PARAMS = dict(compiler_params=pltpu.CompilerParams(
    use_tc_tiling_on_sc=False, needs_layout_passes=False))


def _fill(ref, rows, cols, value):
    v = jnp.full((L,), value, F32)
    nc = cols // L

    def body(k, _):
        i = k // nc
        j = k % nc
        ref[i, pl.ds(j * L, L)] = v
        return 0

    lax.fori_loop(0, rows * nc, body, 0)


@functools.lru_cache(maxsize=None)
def _make_deg(e_pad, na_pad, nb_pad):
    blocks_pt = e_pad // 128 // NS
    nmax = max(na_pad, nb_pad)

    def body(ia_hbm, ib_hbm, ra_hbm, rb_hbm, acc, idxb, ones_v, zb, ssem):
        c = lax.axis_index("c")
        s = lax.axis_index("s")

        _fill(zb, 128, 16, 0.0)
        _fill(ones_v, 128, 16, 1.0)

        def side(idx_hbm, out_hbm, n_pad):
            nblk = n_pad // 128

            def zb_body(i, _):
                b = s + i * NS
                pltpu.sync_copy(zb, acc.at[pl.ds(b * 128, 128)])
                return 0

            lax.fori_loop(0, nblk // NS, zb_body, 0)
            plsc.subcore_barrier()

            pltpu.sync_copy(idx_hbm.at[pl.ds(s * blocks_pt, blocks_pt)],
                            idxb)

            W = 8

            def sc_body(k, _):
                pltpu.async_copy(ones_v, acc.at[idxb.at[k]], ssem, add=True)

                @pl.when(k >= W)
                def _():
                    pltpu.make_async_copy(ones_v, acc.at[idxb.at[0]],
                                          ssem).wait()
                return 0

            lax.fori_loop(0, blocks_pt, sc_body, 0)

            def dr_body(k, _):
                pltpu.make_async_copy(ones_v, acc.at[idxb.at[0]],
                                      ssem).wait()
                return 0

            lax.fori_loop(0, min(W, blocks_pt), dr_body, 0)
            plsc.subcore_barrier()

            def out_body(i, _):
                b = s + i * NS
                pltpu.sync_copy(acc.at[pl.ds(b * 128, 128)], ones_v)
                pltpu.sync_copy(ones_v, out_hbm.at[pl.ds(b * 128, 128)])
                return 0

            lax.fori_loop(0, nblk // NS, out_body, 0)

        @pl.when(c == 0)
        def _():
            side(ia_hbm, ra_hbm, na_pad)

        @pl.when(c == 1)
        def _():
            side(ib_hbm, rb_hbm, nb_pad)

    return pl.kernel(
        body,
        out_type=(jax.ShapeDtypeStruct((na_pad, 16), F32),
                  jax.ShapeDtypeStruct((nb_pad, 16), F32)),
        mesh=_mesh(),
        scratch_types=[
            pltpu.VMEM_SHARED((nmax, 16), F32),
            pltpu.VMEM((blocks_pt, 128), I32),
            pltpu.VMEM((128, 16), F32),
            pltpu.VMEM((128, 16), F32),
            pltpu.SemaphoreType.DMA,
        ],
        **_PARAMS,
    )


def _seg_eb(acc_rows):
    return 64 if acc_rows * HW > 1_500_000 else 128


@functools.lru_cache(maxsize=None)
def _make_seg(e_pad, nsrc_pad, ndst_pad, acc_rows, accw, extra_ws):
    EB = _seg_eb(acc_rows)
    IB = 4 if EB == 64 else 8
    GR = EB // L
    blocks_pt = e_pad // EB // NS
    chunks_pt = blocks_pt // IB
    n_extra = len(extra_ws)
    acc_blk = acc_rows // EB

    def body(*args):
        (table, gidx_hbm, sidx_hbm, deg_hbm), rest = args[:4], args[4:]
        extras = rest[:n_extra]
        rest = rest[n_extra:]
        out_hbm = rest[0]
        acc, ga, gb_, da, db_, rba, rbb, dgb = rest[1:9]
        isa, isb, gsa, gsb = rest[9:13]

        c = lax.axis_index("c")
        s = lax.axis_index("s")
        base = s * blocks_pt
        off = c * nsrc_pad

        _fill(rba, EB, HW, 0.0)

        def z_body(i, _):
            b = s + i * NS

            @pl.when(b < acc_blk)
            def _():
                pltpu.sync_copy(rba, acc.at[pl.ds(b * EB, EB)])
            return 0

        lax.fori_loop(0, (acc_blk + NS - 1) // NS, z_body, 0)
        plsc.subcore_barrier()

        pltpu.sync_copy(
            gidx_hbm.at[pl.ds(base, IB)], ga)
        pltpu.sync_copy(
            sidx_hbm.at[pl.ds(base, IB)], da)

        if chunks_pt > 1:
            pltpu.async_copy(
                gidx_hbm.at[pl.ds(base + IB, IB)], gb_, isb)
            pltpu.async_copy(
                sidx_hbm.at[pl.ds(base + IB, IB)], db_, isb)

        def chunk(ci, gcur, dcur, gnxt, dnxt, my_isem, nxt_isem):
            def p_body(k, _):
                i = k // GR
                j = k % GR
                gcur[i, pl.ds(j * L, L)] = \
                    gcur[i, pl.ds(j * L, L)] + off
                return 0

            lax.fori_loop(0, IB * GR, p_body, 0)

            pltpu.async_copy(table.at[gcur.at[0]], rba, gsa)
            for k in range(IB):
                cur, csem = (rba, gsa) if k % 2 == 0 else (rbb, gsb)
                nxt, nsem = (rbb, gsb) if k % 2 == 0 else (rba, gsa)
                if k + 1 < IB:
                    pltpu.async_copy(table.at[gcur.at[k + 1]], nxt,
                                     nsem)
                pltpu.make_async_copy(table.at[gcur.at[0]], cur,
                                      csem).wait()
                pltpu.sync_copy(cur, acc.at[dcur.at[k]], add=True)

            @pl.when(ci + 2 < chunks_pt)
            def _():
                o = base + (ci + 2) * IB
                pltpu.async_copy(gidx_hbm.at[pl.ds(o, IB)], gcur,
                                 my_isem)
                pltpu.async_copy(sidx_hbm.at[pl.ds(o, IB)], dcur,
                                 my_isem)

            @pl.when(ci + 1 < chunks_pt)
            def _():
                pltpu.make_async_copy(
                    gidx_hbm.at[pl.ds(base, IB)], gnxt,
                    nxt_isem).wait()
                pltpu.make_async_copy(
                    sidx_hbm.at[pl.ds(base, IB)], dnxt,
                    nxt_isem).wait()

        def c_body(ci, _):
            @pl.when(ci % 2 == 0)
            def _():
                chunk(ci, ga, da, gb_, db_, isa, isb)

            @pl.when(ci % 2 == 1)
            def _():
                chunk(ci, gb_, db_, ga, da, isb, isa)
            return 0

        lax.fori_loop(0, chunks_pt, c_body, 0)
        plsc.subcore_barrier()

        out_off = c * ndst_pad

        def o_body(i, _):
            b = s + i * NS

            @pl.when(b < acc_blk)
            def _():
                r0 = b * EB
                pltpu.sync_copy(acc.at[pl.ds(r0, EB)], rba)
                pltpu.sync_copy(deg_hbm.at[pl.ds(r0, EB)], dgb)

                def row_body(ri, _):
                    dv = dgb[ri, pl.ds(0, L)]
                    sc = jnp.full((L,), accw, F32) \
                        / jnp.maximum(dv, 1.0)
                    for j in range(4):
                        rba[ri, pl.ds(j * L, L)] = \
                            rba[ri, pl.ds(j * L, L)] * sc
                    return 0

                lax.fori_loop(0, EB, row_body, 0)

                for t in range(n_extra):
                    pltpu.sync_copy(
                        extras[t].at[pl.ds(out_off + r0, EB)], rbb)

                    def e_body(k, _):
                        ri = k // 4
                        j = k % 4
                        rba[ri, pl.ds(j * L, L)] = \
                            rba[ri, pl.ds(j * L, L)] \
                            + rbb[ri, pl.ds(j * L, L)] * extra_ws[t]
                        return 0

                    lax.fori_loop(0, EB * 4, e_body, 0)

                pltpu.sync_copy(rba,
                                out_hbm.at[pl.ds(out_off + r0, EB)])
            return 0

        lax.fori_loop(0, (acc_blk + NS - 1) // NS, o_body, 0)

    def body2(*args):
        body(*args)

    scratch = [
        pltpu.VMEM_SHARED((acc_rows, HW), F32),
        pltpu.VMEM((IB, EB), I32),
        pltpu.VMEM((IB, EB), I32),
        pltpu.VMEM((IB, EB), I32),
        pltpu.VMEM((IB, EB), I32),
        pltpu.VMEM((EB, HW), F32),
        pltpu.VMEM((EB, HW), F32),
        pltpu.VMEM((EB, 16), F32),
        pltpu.SemaphoreType.DMA,
        pltpu.SemaphoreType.DMA,
        pltpu.SemaphoreType.DMA,
        pltpu.SemaphoreType.DMA,
    ]

    return pl.kernel(
        body2,
        out_type=jax.ShapeDtypeStruct((2 * ndst_pad, HW), F32),
        mesh=_mesh(),
        scratch_types=scratch,
        **_PARAMS,
    )


GB = 8


@functools.lru_cache(maxsize=None)
def _make_lit(n_seq, ntok_pad, nq_pad, t_real, t_pad, alpha, extra_ws,
              tok_full=False, full_out=False):
    n_extra = len(extra_ws)
    nblk = (n_seq + L - 1) // L
    ngrp = (nblk + GB - 1) // GB
    gseq = GB * L
    T = t_real
    NW = NC * NS
    nhb = 1 if tok_full else 2

    def body(*args):
        (tok, q_hbm, tidx_hbm), rest = args[:3], args[3:]
        extras = rest[:n_extra]
        rest = rest[n_extra:]
        out_hbm = rest[0]
        rest = rest[1:]
        if full_out:
            fout_hbm = rest[0]
            rest = rest[1:]
        ibuf = rest[0]
        qb = rest[1:3]
        eb = rest[3:3 + 2 * n_extra]
        rest = rest[3 + 2 * n_extra:]
        ob = rest[0:2]
        rest = rest[2:]
        if full_out:
            ofull = rest[0]
            rest = rest[1:]
        if tok_full:
            tba = rest[0:1]
            tbb = rest[1:2]
            rest = rest[2:]
            stA = stB = None
        else:
            stA = rest[0:2]
            stB = rest[2:4]
            tba = rest[4:6]
            tbb = rest[6:8]
            rest = rest[8:]
        gsa, gsb = rest[0], rest[1]

        c = lax.axis_index("c")
        s = lax.axis_index("s")
        wid = s * NC + c
        lane = lax.iota(I32, L)

        def tget(tq, t, u):
            if tok_full:
                return tq[0][t, pl.ds(u * L, L)]
            return tq[u // 4][t, pl.ds((u % 4) * L, L)]

        def do_group(g):
            s0 = g * gseq
            pltpu.sync_copy(tidx_hbm.at[pl.ds(s0 * t_pad, gseq * t_pad)],
                            ibuf)
            for h in range(2):
                r = h * nq_pad + s0
                pltpu.sync_copy(q_hbm.at[pl.ds(r, gseq)], qb[h])
                for t in range(n_extra):
                    pltpu.sync_copy(extras[t].at[pl.ds(r, gseq)],
                                    eb[t * 2 + h])

            def fire(sq, strips, bufs, sem):
                o = sq * t_pad
                if tok_full:
                    pltpu.async_copy(
                        tok.at[ibuf.at[pl.ds(o, T)]], bufs[0], sem)
                else:
                    nvr = t_pad // L
                    for h in range(2):
                        for j in range(nvr):
                            strips[h][pl.ds(j * L, L)] = \
                                ibuf[pl.ds(o + j * L, L)] + (h * ntok_pad)
                    for h in range(2):
                        pltpu.async_copy(
                            tok.at[strips[h].at[pl.ds(0, T)]], bufs[h],
                            sem)

            def drain(bufs, sem):
                if tok_full:
                    pltpu.make_async_copy(
                        tok.at[ibuf.at[pl.ds(0, T)]], bufs[0], sem).wait()
                else:
                    for h in range(2):
                        pltpu.make_async_copy(
                            tok.at[stA[h].at[pl.ds(0, T)]], bufs[h],
                            sem).wait()

            fire(0, stA, tba, gsa)

            def seq(sq, tq, cur_sem, nstrips, nbufs, nxt_sem):
                @pl.when(sq + 1 < gseq)
                def _():
                    fire(sq + 1, nstrips, nbufs, nxt_sem)

                drain(tq, cur_sem)

                qv = [qb[h][sq, pl.ds(j * L, L)]
                      for h in range(2) for j in range(4)]

                sc0 = jnp.full((L,), -1e30, F32)
                sc1 = jnp.full((L,), -1e30, F32)
                for t in range(T):
                    sv = qv[0] * tget(tq, t, 0)
                    for u in range(1, 8):
                        sv = sv + qv[u] * tget(tq, t, u)
                    sdot = jnp.sum(sv) * float(DIM ** -0.5)
                    msk = lane == (t % L)
                    if t < L:
                        sc0 = jnp.where(msk, sdot, sc0)
                    else:
                        sc1 = jnp.where(msk, sdot, sc1)

                if T > L:
                    m = jnp.maximum(jnp.max(sc0), jnp.max(sc1))
                    w0 = jnp.exp(sc0 - m)
                    w1 = jnp.exp(sc1 - m)
                    denom = jnp.sum(w0) + jnp.sum(w1)
                else:
                    m = jnp.max(sc0)
                    w0 = jnp.exp(sc0 - m)
                    w1 = w0
                    denom = jnp.sum(w0)

                rr = jnp.full((L,), 1.0 - alpha, F32) \
                    / (jnp.full((L,), 0.0, F32) + denom)

                accv = [jnp.zeros((L,), F32) for _ in range(8)]
                for t in range(T):
                    wsrc = w0 if t < L else w1
                    wt = wsrc.at[jnp.full((L,), t % L, I32)].get(
                        mode="promise_in_bounds")
                    for u in range(8):
                        accv[u] = accv[u] + wt * tget(tq, t, u)

                for u in range(8):
                    h, j = u // 4, u % 4
                    v = alpha * qv[u] + rr * accv[u]
                    for t in range(n_extra):
                        v = v + eb[t * 2 + h][sq, pl.ds(j * L, L)] \
                            * extra_ws[t]
                    ob[h][sq, pl.ds(j * L, L)] = v
                    if full_out:
                        ofull[sq, pl.ds(u * L, L)] = v

            def s_body(sq, _):
                @pl.when(sq % 2 == 0)
                def _():
                    seq(sq, tba, gsa, stB, tbb, gsb)

                @pl.when(sq % 2 == 1)
                def _():
                    seq(sq, tbb, gsb, stA, tba, gsa)
                return 0

            lax.fori_loop(0, gseq, s_body, 0)

            for h in range(2):
                pltpu.sync_copy(ob[h],
                                out_hbm.at[pl.ds(h * nq_pad + s0, gseq)])
            if full_out:
                pltpu.sync_copy(ofull, fout_hbm.at[pl.ds(s0, gseq)])

        def g_body(i, _):
            g = wid + i * NW

            @pl.when(g < ngrp)
            def _():
                do_group(g)
            return 0

        lax.fori_loop(0, (ngrp + NW - 1) // NW, g_body, 0)

    scratch = [pltpu.VMEM((gseq * t_pad,), I32)]
    scratch += [pltpu.VMEM((gseq, HW), F32) for _ in range(2)]
    scratch += [pltpu.VMEM((gseq, HW), F32) for _ in range(2 * n_extra)]
    scratch += [pltpu.VMEM((gseq, HW), F32) for _ in range(2)]
    if full_out:
        scratch += [pltpu.VMEM((gseq, DIM), F32)]
    if tok_full:
        scratch += [pltpu.VMEM((T, DIM), F32) for _ in range(2)]
    else:
        scratch += [pltpu.VMEM((t_pad,), I32) for _ in range(4)]
        scratch += [pltpu.VMEM((T, HW), F32) for _ in range(4)]
    scratch += [pltpu.SemaphoreType.DMA, pltpu.SemaphoreType.DMA]

    out_type = jax.ShapeDtypeStruct((2 * nq_pad, HW), F32)
    if full_out:
        out_type = (out_type, jax.ShapeDtypeStruct((nq_pad, DIM), F32))

    return pl.kernel(
        body,
        out_type=out_type,
        mesh=_mesh(),
        scratch_types=scratch,
        **_PARAMS,
    )


def _planarize(x, n_pad):
    n = x.shape[0]
    x = jnp.pad(x, ((0, n_pad - n), (0, 0)))
    return x.reshape(n_pad, 2, HW).transpose(1, 0, 2).reshape(
        2 * n_pad, HW)


def _unplanarize(x, n, n_pad):
    return x.reshape(2, n_pad, HW)[:, :n, :].transpose(1, 0, 2).reshape(
        n, DIM)


def _pad_edges(col, e_pad, pad_base, spread, eb):
    e = col.shape[0]
    if e < e_pad:
        pad = pad_base + (jnp.arange(e_pad - e, dtype=col.dtype) % spread)
        col = jnp.concatenate([col, pad])
    return col.reshape(e_pad // eb, eb)


def _prep_graph(edges, na, nb):
    e = edges.shape[1]
    ep = _epad(e)
    a = edges[0].astype(I32)
    b = edges[1].astype(I32)
    out = {"ep": ep}
    for eb in (64, 128):
        out[f"a_g{eb}"] = _pad_edges(a, ep, 0, min(na, 1024), eb)
        out[f"b_g{eb}"] = _pad_edges(b, ep, 0, min(nb, 1024), eb)
        out[f"a_s{eb}"] = _pad_edges(a, ep, na, 64, eb)
        out[f"b_s{eb}"] = _pad_edges(b, ep, nb, 64, eb)
    return out


def _pad_tokens(tidx, t_pad, s_pad):
    s, t = tidx.shape
    out = jnp.pad(tidx.astype(I32), ((0, s_pad - s), (0, t_pad - t)))
    return out.reshape(s_pad * t_pad)


def kernel(user_embedding, item_embedding, bundle_embedding, ui_edge_index,
           ub_edge_index, bi_edge_index, ub_token_idx, bi_token_idx):
    NU, NI, NB = NP_USER, NP_ITEM, NP_BUNDLE

    u0 = _planarize(user_embedding, NU)
    i0 = _planarize(item_embedding, NI)
    b0 = _planarize(bundle_embedding, NB)

    ui = _prep_graph(ui_edge_index, N_USER, N_ITEM)
    ub = _prep_graph(ub_edge_index, N_USER, N_BUNDLE)
    bi = _prep_graph(bi_edge_index, N_BUNDLE, N_ITEM)

    sp_u = ((N_USER + 127) // 128) * 128
    sp_b = ((N_BUNDLE + 127) // 128) * 128
    ub_tidx = _pad_tokens(ub_token_idx, 32, sp_u)
    bi_tidx = _pad_tokens(bi_token_idx, 16, sp_b)

    d_u_ui, d_i_ui = _make_deg(ui["ep"], NU, NI)(ui["a_s128"],
                                                 ui["b_s128"])
    d_u_ub, d_b_ub = _make_deg(ub["ep"], NU, NB)(ub["a_s128"],
                                                 ub["b_s128"])
    d_b_bi, d_i_bi = _make_deg(bi["ep"], NB, NI)(bi["a_s128"],
                                                 bi["b_s128"])

    def seg(table, gr, gcol, scol, deg, nsp, ndst, ndp, accw=1.0,
            extras=()):
        ws = tuple(float(w) for (_, w) in extras)
        arrs = tuple(a for (a, _) in extras)
        acc_rows = _accrows(ndst)
        eb = _seg_eb(acc_rows)
        g2 = gr[f"{gcol}_g{eb}"]
        s2 = gr[f"{scol}_s{eb}"]
        return _make_seg(gr["ep"], nsp, ndp, acc_rows, float(accw), ws)(
            table, g2, s2, deg, *arrs)

    u1 = seg(i0, ui, "b", "a", d_u_ui, NI, N_USER, NU)
    i1 = seg(u0, ui, "a", "b", d_i_ui, NU, N_ITEM, NI)
    u2 = seg(i1, ui, "b", "a", d_u_ui, NI, N_USER, NU)
    i2 = seg(u1, ui, "a", "b", d_i_ui, NU, N_ITEM, NI)
    u_cf = seg(i2, ui, "b", "a", d_u_ui, NI, N_USER, NU,
               accw=0.25, extras=((u0, 0.25), (u1, 0.25), (u2, 0.25)))
    i_cf = seg(u2, ui, "a", "b", d_i_ui, NU, N_ITEM, NI,
               accw=0.25, extras=((i0, 0.25), (i1, 0.25), (i2, 0.25)))

    u_f, i_f, b_f = u0, i0, b0
    for _ in range(3):
        u_b2u = seg(b_f, ub, "b", "a", d_u_ub, NB, N_USER, NU)
        b_u2b = seg(u_f, ub, "a", "b", d_b_ub, NU, N_BUNDLE, NB)
        b_i2b = seg(i_f, bi, "b", "a", d_b_bi, NI, N_BUNDLE, NB)
        i_new = seg(b_f, bi, "a", "b", d_i_bi, NB, N_ITEM, NI,
                    extras=((i_cf, 1.0),))
        b_new = _make_lit(N_BUNDLE, NI, NB, 5, 16, 0.5, (1.0,))(
            i_new, b_i2b, bi_tidx, b_u2b)
        u_new = _make_lit(N_USER, NB, NU, 30, 32, 0.5, (1.0,))(
            b_new, u_b2u, ub_tidx, u_cf)
        u_f, i_f, b_f = u_new, i_new, b_new

    return (_unplanarize(u_f, N_USER, NU),
            _unplanarize(i_f, N_ITEM, NI),
            _unplanarize(b_f, N_BUNDLE, NB))


# speedup vs baseline: 1.0272x; 1.0026x over previous
import functools

import jax
import jax.numpy as jnp
from jax import lax
from jax.experimental import pallas as pl
from jax.experimental.pallas import tpu as pltpu
from jax.experimental.pallas import tpu_sc as plsc

N_USER = 20000
N_ITEM = 30000
N_BUNDLE = 10000
DIM = 128
HW = 64
NC = 2
NS = 16
L = 16

F32 = jnp.float32
I32 = jnp.int32


def _rpad(n):
    return ((n + 2047) // 2048) * 2048


NP_USER = _rpad(N_USER)
NP_ITEM = _rpad(N_ITEM)
NP_BUNDLE = _rpad(N_BUNDLE)


def _accrows(n):
    return ((n + 64 + 127) // 128) * 128


def _epad(e):
    return ((e + 16383) // 16384) * 16384


def _mesh():
    return plsc.VectorSubcoreMesh(
        core_axis_name="c", subcore_axis_name="s", num_cores=NC,
        num_subcores=NS)


_PARAMS = dict(compiler_params=pltpu.CompilerParams(
    use_tc_tiling_on_sc=False, needs_layout_passes=False))


def _fill(ref, rows, cols, value):
    v = jnp.full((L,), value, F32)
    nc = cols // L

    def body(k, _):
        i = k // nc
        j = k % nc
        ref[i, pl.ds(j * L, L)] = v
        return 0

    lax.fori_loop(0, rows * nc, body, 0)


@functools.lru_cache(maxsize=None)
def _make_deg(e_pad, na_pad, nb_pad):
    blocks_pt = e_pad // 128 // NS
    nmax = max(na_pad, nb_pad)

    def body(ia_hbm, ib_hbm, ra_hbm, rb_hbm, acc, idxb, ones_v, zb, ssem):
        c = lax.axis_index("c")
        s = lax.axis_index("s")

        _fill(zb, 128, 16, 0.0)
        _fill(ones_v, 128, 16, 1.0)

        def side(idx_hbm, out_hbm, n_pad):
            nblk = n_pad // 128

            def zb_body(i, _):
                b = s + i * NS
                pltpu.sync_copy(zb, acc.at[pl.ds(b * 128, 128)])
                return 0

            lax.fori_loop(0, nblk // NS, zb_body, 0)
            plsc.subcore_barrier()

            pltpu.sync_copy(idx_hbm.at[pl.ds(s * blocks_pt, blocks_pt)],
                            idxb)

            W = 8

            def sc_body(k, _):
                pltpu.async_copy(ones_v, acc.at[idxb.at[k]], ssem, add=True)

                @pl.when(k >= W)
                def _():
                    pltpu.make_async_copy(ones_v, acc.at[idxb.at[0]],
                                          ssem).wait()
                return 0

            lax.fori_loop(0, blocks_pt, sc_body, 0)

            def dr_body(k, _):
                pltpu.make_async_copy(ones_v, acc.at[idxb.at[0]],
                                      ssem).wait()
                return 0

            lax.fori_loop(0, min(W, blocks_pt), dr_body, 0)
            plsc.subcore_barrier()

            def out_body(i, _):
                b = s + i * NS
                pltpu.sync_copy(acc.at[pl.ds(b * 128, 128)], ones_v)
                pltpu.sync_copy(ones_v, out_hbm.at[pl.ds(b * 128, 128)])
                return 0

            lax.fori_loop(0, nblk // NS, out_body, 0)

        @pl.when(c == 0)
        def _():
            side(ia_hbm, ra_hbm, na_pad)

        @pl.when(c == 1)
        def _():
            side(ib_hbm, rb_hbm, nb_pad)

    return pl.kernel(
        body,
        out_type=(jax.ShapeDtypeStruct((na_pad, 16), F32),
                  jax.ShapeDtypeStruct((nb_pad, 16), F32)),
        mesh=_mesh(),
        scratch_types=[
            pltpu.VMEM_SHARED((nmax, 16), F32),
            pltpu.VMEM((blocks_pt, 128), I32),
            pltpu.VMEM((128, 16), F32),
            pltpu.VMEM((128, 16), F32),
            pltpu.SemaphoreType.DMA,
        ],
        **_PARAMS,
    )


def _seg_eb(acc_rows):
    return 64 if acc_rows * HW > 1_500_000 else 128


@functools.lru_cache(maxsize=None)
def _make_seg(e_pad, nsrc_pad, ndst_pad, acc_rows, accw, extra_ws):
    EB = _seg_eb(acc_rows)
    IB = 4 if EB == 64 else 8
    GR = EB // L
    blocks_pt = e_pad // EB // NS
    chunks_pt = blocks_pt // IB
    n_extra = len(extra_ws)
    acc_blk = acc_rows // EB

    def body(*args):
        (table, gidx_hbm, sidx_hbm, deg_hbm), rest = args[:4], args[4:]
        extras = rest[:n_extra]
        rest = rest[n_extra:]
        out_hbm = rest[0]
        acc, ga, gb_, da, db_, rba, rbb, dgb = rest[1:9]
        isa, isb, gsa, gsb, sca, scb = rest[9:15]

        c = lax.axis_index("c")
        s = lax.axis_index("s")
        base = s * blocks_pt
        off = c * nsrc_pad

        _fill(rba, EB, HW, 0.0)

        def z_body(i, _):
            b = s + i * NS

            @pl.when(b < acc_blk)
            def _():
                pltpu.sync_copy(rba, acc.at[pl.ds(b * EB, EB)])
            return 0

        lax.fori_loop(0, (acc_blk + NS - 1) // NS, z_body, 0)
        plsc.subcore_barrier()

        pltpu.sync_copy(
            gidx_hbm.at[pl.ds(base, IB)], ga)
        pltpu.sync_copy(
            sidx_hbm.at[pl.ds(base, IB)], da)

        if chunks_pt > 1:
            pltpu.async_copy(
                gidx_hbm.at[pl.ds(base + IB, IB)], gb_, isb)
            pltpu.async_copy(
                sidx_hbm.at[pl.ds(base + IB, IB)], db_, isb)

        def chunk(ci, gcur, dcur, gnxt, dnxt, my_isem, nxt_isem):
            def p_body(k, _):
                i = k // GR
                j = k % GR
                gcur[i, pl.ds(j * L, L)] = \
                    gcur[i, pl.ds(j * L, L)] + off
                return 0

            lax.fori_loop(0, IB * GR, p_body, 0)

            pltpu.async_copy(table.at[gcur.at[0]], rba, gsa)
            for k in range(IB):
                cur, csem, ssc = (rba, gsa, sca) if k % 2 == 0 \
                    else (rbb, gsb, scb)
                nxt, nsem, nsc = (rbb, gsb, scb) if k % 2 == 0 \
                    else (rba, gsa, sca)
                if k + 1 < IB:
                    if k >= 1:
                        pltpu.make_async_copy(nxt, acc.at[dcur.at[0]],
                                              nsc).wait()
                    pltpu.async_copy(table.at[gcur.at[k + 1]], nxt,
                                     nsem)
                pltpu.make_async_copy(table.at[gcur.at[0]], cur,
                                      csem).wait()
                pltpu.async_copy(cur, acc.at[dcur.at[k]], ssc, add=True)

            pltpu.make_async_copy(rba, acc.at[dcur.at[0]], sca).wait()
            pltpu.make_async_copy(rbb, acc.at[dcur.at[0]], scb).wait()

            @pl.when(ci + 2 < chunks_pt)
            def _():
                o = base + (ci + 2) * IB
                pltpu.async_copy(gidx_hbm.at[pl.ds(o, IB)], gcur,
                                 my_isem)
                pltpu.async_copy(sidx_hbm.at[pl.ds(o, IB)], dcur,
                                 my_isem)

            @pl.when(ci + 1 < chunks_pt)
            def _():
                pltpu.make_async_copy(
                    gidx_hbm.at[pl.ds(base, IB)], gnxt,
                    nxt_isem).wait()
                pltpu.make_async_copy(
                    sidx_hbm.at[pl.ds(base, IB)], dnxt,
                    nxt_isem).wait()

        def c_body(ci, _):
            @pl.when(ci % 2 == 0)
            def _():
                chunk(ci, ga, da, gb_, db_, isa, isb)

            @pl.when(ci % 2 == 1)
            def _():
                chunk(ci, gb_, db_, ga, da, isb, isa)
            return 0

        lax.fori_loop(0, chunks_pt, c_body, 0)
        plsc.subcore_barrier()

        out_off = c * ndst_pad

        def o_body(i, _):
            b = s + i * NS

            @pl.when(b < acc_blk)
            def _():
                r0 = b * EB
                pltpu.sync_copy(acc.at[pl.ds(r0, EB)], rba)
                pltpu.sync_copy(deg_hbm.at[pl.ds(r0, EB)], dgb)

                def row_body(ri, _):
                    dv = dgb[ri, pl.ds(0, L)]
                    sc = jnp.full((L,), accw, F32) \
                        / jnp.maximum(dv, 1.0)
                    for j in range(4):
                        rba[ri, pl.ds(j * L, L)] = \
                            rba[ri, pl.ds(j * L, L)] * sc
                    return 0

                lax.fori_loop(0, EB, row_body, 0)

                for t in range(n_extra):
                    pltpu.sync_copy(
                        extras[t].at[pl.ds(out_off + r0, EB)], rbb)

                    def e_body(k, _):
                        ri = k // 4
                        j = k % 4
                        rba[ri, pl.ds(j * L, L)] = \
                            rba[ri, pl.ds(j * L, L)] \
                            + rbb[ri, pl.ds(j * L, L)] * extra_ws[t]
                        return 0

                    lax.fori_loop(0, EB * 4, e_body, 0)

                pltpu.sync_copy(rba,
                                out_hbm.at[pl.ds(out_off + r0, EB)])
            return 0

        lax.fori_loop(0, (acc_blk + NS - 1) // NS, o_body, 0)

    def body2(*args):
        body(*args)

    scratch = [
        pltpu.VMEM_SHARED((acc_rows, HW), F32),
        pltpu.VMEM((IB, EB), I32),
        pltpu.VMEM((IB, EB), I32),
        pltpu.VMEM((IB, EB), I32),
        pltpu.VMEM((IB, EB), I32),
        pltpu.VMEM((EB, HW), F32),
        pltpu.VMEM((EB, HW), F32),
        pltpu.VMEM((EB, 16), F32),
        pltpu.SemaphoreType.DMA,
        pltpu.SemaphoreType.DMA,
        pltpu.SemaphoreType.DMA,
        pltpu.SemaphoreType.DMA,
        pltpu.SemaphoreType.DMA,
        pltpu.SemaphoreType.DMA,
    ]

    return pl.kernel(
        body2,
        out_type=jax.ShapeDtypeStruct((2 * ndst_pad, HW), F32),
        mesh=_mesh(),
        scratch_types=scratch,
        **_PARAMS,
    )


GB = 8


@functools.lru_cache(maxsize=None)
def _make_lit(n_seq, ntok_pad, nq_pad, t_real, t_pad, alpha, extra_ws,
              tok_full=False, full_out=False):
    n_extra = len(extra_ws)
    nblk = (n_seq + L - 1) // L
    ngrp = (nblk + GB - 1) // GB
    gseq = GB * L
    T = t_real
    NW = NC * NS
    nhb = 1 if tok_full else 2

    def body(*args):
        (tok, q_hbm, tidx_hbm), rest = args[:3], args[3:]
        extras = rest[:n_extra]
        rest = rest[n_extra:]
        out_hbm = rest[0]
        rest = rest[1:]
        if full_out:
            fout_hbm = rest[0]
            rest = rest[1:]
        ibuf = rest[0]
        qb = rest[1:3]
        eb = rest[3:3 + 2 * n_extra]
        rest = rest[3 + 2 * n_extra:]
        ob = rest[0:2]
        rest = rest[2:]
        if full_out:
            ofull = rest[0]
            rest = rest[1:]
        if tok_full:
            tba = rest[0:1]
            tbb = rest[1:2]
            rest = rest[2:]
            stA = stB = None
        else:
            stA = rest[0:2]
            stB = rest[2:4]
            tba = rest[4:6]
            tbb = rest[6:8]
            rest = rest[8:]
        gsa, gsb = rest[0], rest[1]

        c = lax.axis_index("c")
        s = lax.axis_index("s")
        wid = s * NC + c
        lane = lax.iota(I32, L)

        def tget(tq, t, u):
            if tok_full:
                return tq[0][t, pl.ds(u * L, L)]
            return tq[u // 4][t, pl.ds((u % 4) * L, L)]

        def do_group(g):
            s0 = g * gseq
            pltpu.sync_copy(tidx_hbm.at[pl.ds(s0 * t_pad, gseq * t_pad)],
                            ibuf)
            for h in range(2):
                r = h * nq_pad + s0
                pltpu.sync_copy(q_hbm.at[pl.ds(r, gseq)], qb[h])
                for t in range(n_extra):
                    pltpu.sync_copy(extras[t].at[pl.ds(r, gseq)],
                                    eb[t * 2 + h])

            def fire(sq, strips, bufs, sem):
                o = sq * t_pad
                if tok_full:
                    pltpu.async_copy(
                        tok.at[ibuf.at[pl.ds(o, T)]], bufs[0], sem)
                else:
                    nvr = t_pad // L
                    for h in range(2):
                        for j in range(nvr):
                            strips[h][pl.ds(j * L, L)] = \
                                ibuf[pl.ds(o + j * L, L)] + (h * ntok_pad)
                    for h in range(2):
                        pltpu.async_copy(
                            tok.at[strips[h].at[pl.ds(0, T)]], bufs[h],
                            sem)

            def drain(bufs, sem):
                if tok_full:
                    pltpu.make_async_copy(
                        tok.at[ibuf.at[pl.ds(0, T)]], bufs[0], sem).wait()
                else:
                    for h in range(2):
                        pltpu.make_async_copy(
                            tok.at[stA[h].at[pl.ds(0, T)]], bufs[h],
                            sem).wait()

            fire(0, stA, tba, gsa)

            def seq(sq, tq, cur_sem, nstrips, nbufs, nxt_sem):
                @pl.when(sq + 1 < gseq)
                def _():
                    fire(sq + 1, nstrips, nbufs, nxt_sem)

                drain(tq, cur_sem)

                qv = [qb[h][sq, pl.ds(j * L, L)]
                      for h in range(2) for j in range(4)]

                sc0 = jnp.full((L,), -1e30, F32)
                sc1 = jnp.full((L,), -1e30, F32)
                for t in range(T):
                    sv = qv[0] * tget(tq, t, 0)
                    for u in range(1, 8):
                        sv = sv + qv[u] * tget(tq, t, u)
                    sdot = jnp.sum(sv) * float(DIM ** -0.5)
                    msk = lane == (t % L)
                    if t < L:
                        sc0 = jnp.where(msk, sdot, sc0)
                    else:
                        sc1 = jnp.where(msk, sdot, sc1)

                if T > L:
                    m = jnp.maximum(jnp.max(sc0), jnp.max(sc1))
                    w0 = jnp.exp(sc0 - m)
                    w1 = jnp.exp(sc1 - m)
                    denom = jnp.sum(w0) + jnp.sum(w1)
                else:
                    m = jnp.max(sc0)
                    w0 = jnp.exp(sc0 - m)
                    w1 = w0
                    denom = jnp.sum(w0)

                rr = jnp.full((L,), 1.0 - alpha, F32) \
                    / (jnp.full((L,), 0.0, F32) + denom)

                accv = [jnp.zeros((L,), F32) for _ in range(8)]
                for t in range(T):
                    wsrc = w0 if t < L else w1
                    wt = wsrc.at[jnp.full((L,), t % L, I32)].get(
                        mode="promise_in_bounds")
                    for u in range(8):
                        accv[u] = accv[u] + wt * tget(tq, t, u)

                for u in range(8):
                    h, j = u // 4, u % 4
                    v = alpha * qv[u] + rr * accv[u]
                    for t in range(n_extra):
                        v = v + eb[t * 2 + h][sq, pl.ds(j * L, L)] \
                            * extra_ws[t]
                    ob[h][sq, pl.ds(j * L, L)] = v
                    if full_out:
                        ofull[sq, pl.ds(u * L, L)] = v

            def s_body(sq, _):
                @pl.when(sq % 2 == 0)
                def _():
                    seq(sq, tba, gsa, stB, tbb, gsb)

                @pl.when(sq % 2 == 1)
                def _():
                    seq(sq, tbb, gsb, stA, tba, gsa)
                return 0

            lax.fori_loop(0, gseq, s_body, 0)

            for h in range(2):
                pltpu.sync_copy(ob[h],
                                out_hbm.at[pl.ds(h * nq_pad + s0, gseq)])
            if full_out:
                pltpu.sync_copy(ofull, fout_hbm.at[pl.ds(s0, gseq)])

        def g_body(i, _):
            g = wid + i * NW

            @pl.when(g < ngrp)
            def _():
                do_group(g)
            return 0

        lax.fori_loop(0, (ngrp + NW - 1) // NW, g_body, 0)

    scratch = [pltpu.VMEM((gseq * t_pad,), I32)]
    scratch += [pltpu.VMEM((gseq, HW), F32) for _ in range(2)]
    scratch += [pltpu.VMEM((gseq, HW), F32) for _ in range(2 * n_extra)]
    scratch += [pltpu.VMEM((gseq, HW), F32) for _ in range(2)]
    if full_out:
        scratch += [pltpu.VMEM((gseq, DIM), F32)]
    if tok_full:
        scratch += [pltpu.VMEM((T, DIM), F32) for _ in range(2)]
    else:
        scratch += [pltpu.VMEM((t_pad,), I32) for _ in range(4)]
        scratch += [pltpu.VMEM((T, HW), F32) for _ in range(4)]
    scratch += [pltpu.SemaphoreType.DMA, pltpu.SemaphoreType.DMA]

    out_type = jax.ShapeDtypeStruct((2 * nq_pad, HW), F32)
    if full_out:
        out_type = (out_type, jax.ShapeDtypeStruct((nq_pad, DIM), F32))

    return pl.kernel(
        body,
        out_type=out_type,
        mesh=_mesh(),
        scratch_types=scratch,
        **_PARAMS,
    )


def _planarize(x, n_pad):
    n = x.shape[0]
    x = jnp.pad(x, ((0, n_pad - n), (0, 0)))
    return x.reshape(n_pad, 2, HW).transpose(1, 0, 2).reshape(
        2 * n_pad, HW)


def _unplanarize(x, n, n_pad):
    return x.reshape(2, n_pad, HW)[:, :n, :].transpose(1, 0, 2).reshape(
        n, DIM)


def _pad_edges(col, e_pad, pad_base, spread, eb):
    e = col.shape[0]
    if e < e_pad:
        pad = pad_base + (jnp.arange(e_pad - e, dtype=col.dtype) % spread)
        col = jnp.concatenate([col, pad])
    return col.reshape(e_pad // eb, eb)


def _prep_graph(edges, na, nb):
    e = edges.shape[1]
    ep = _epad(e)
    a = edges[0].astype(I32)
    b = edges[1].astype(I32)
    out = {"ep": ep}
    for eb in (64, 128):
        out[f"a_g{eb}"] = _pad_edges(a, ep, 0, min(na, 1024), eb)
        out[f"b_g{eb}"] = _pad_edges(b, ep, 0, min(nb, 1024), eb)
        out[f"a_s{eb}"] = _pad_edges(a, ep, na, 64, eb)
        out[f"b_s{eb}"] = _pad_edges(b, ep, nb, 64, eb)
    return out


def _pad_tokens(tidx, t_pad, s_pad):
    s, t = tidx.shape
    out = jnp.pad(tidx.astype(I32), ((0, s_pad - s), (0, t_pad - t)))
    return out.reshape(s_pad * t_pad)


def kernel(user_embedding, item_embedding, bundle_embedding, ui_edge_index,
           ub_edge_index, bi_edge_index, ub_token_idx, bi_token_idx):
    NU, NI, NB = NP_USER, NP_ITEM, NP_BUNDLE

    u0 = _planarize(user_embedding, NU)
    i0 = _planarize(item_embedding, NI)
    b0 = _planarize(bundle_embedding, NB)

    ui = _prep_graph(ui_edge_index, N_USER, N_ITEM)
    ub = _prep_graph(ub_edge_index, N_USER, N_BUNDLE)
    bi = _prep_graph(bi_edge_index, N_BUNDLE, N_ITEM)

    sp_u = ((N_USER + 127) // 128) * 128
    sp_b = ((N_BUNDLE + 127) // 128) * 128
    ub_tidx = _pad_tokens(ub_token_idx, 32, sp_u)
    bi_tidx = _pad_tokens(bi_token_idx, 16, sp_b)

    d_u_ui, d_i_ui = _make_deg(ui["ep"], NU, NI)(ui["a_s128"],
                                                 ui["b_s128"])
    d_u_ub, d_b_ub = _make_deg(ub["ep"], NU, NB)(ub["a_s128"],
                                                 ub["b_s128"])
    d_b_bi, d_i_bi = _make_deg(bi["ep"], NB, NI)(bi["a_s128"],
                                                 bi["b_s128"])

    def seg(table, gr, gcol, scol, deg, nsp, ndst, ndp, accw=1.0,
            extras=()):
        ws = tuple(float(w) for (_, w) in extras)
        arrs = tuple(a for (a, _) in extras)
        acc_rows = _accrows(ndst)
        eb = _seg_eb(acc_rows)
        g2 = gr[f"{gcol}_g{eb}"]
        s2 = gr[f"{scol}_s{eb}"]
        return _make_seg(gr["ep"], nsp, ndp, acc_rows, float(accw), ws)(
            table, g2, s2, deg, *arrs)

    u1 = seg(i0, ui, "b", "a", d_u_ui, NI, N_USER, NU)
    i1 = seg(u0, ui, "a", "b", d_i_ui, NU, N_ITEM, NI)
    u2 = seg(i1, ui, "b", "a", d_u_ui, NI, N_USER, NU)
    i2 = seg(u1, ui, "a", "b", d_i_ui, NU, N_ITEM, NI)
    u_cf = seg(i2, ui, "b", "a", d_u_ui, NI, N_USER, NU,
               accw=0.25, extras=((u0, 0.25), (u1, 0.25), (u2, 0.25)))
    i_cf = seg(u2, ui, "a", "b", d_i_ui, NU, N_ITEM, NI,
               accw=0.25, extras=((i0, 0.25), (i1, 0.25), (i2, 0.25)))

    u_f, i_f, b_f = u0, i0, b0
    for _ in range(3):
        u_b2u = seg(b_f, ub, "b", "a", d_u_ub, NB, N_USER, NU)
        b_u2b = seg(u_f, ub, "a", "b", d_b_ub, NU, N_BUNDLE, NB)
        b_i2b = seg(i_f, bi, "b", "a", d_b_bi, NI, N_BUNDLE, NB)
        i_new = seg(b_f, bi, "a", "b", d_i_bi, NB, N_ITEM, NI,
                    extras=((i_cf, 1.0),))
        b_new = _make_lit(N_BUNDLE, NI, NB, 5, 16, 0.5, (1.0,))(
            i_new, b_i2b, bi_tidx, b_u2b)
        u_new = _make_lit(N_USER, NB, NU, 30, 32, 0.5, (1.0,))(
            b_new, u_b2u, ub_tidx, u_cf)
        u_f, i_f, b_f = u_new, i_new, b_new

    return (_unplanarize(u_f, N_USER, NU),
            _unplanarize(i_f, N_ITEM, NI),
            _unplanarize(b_f, N_BUNDLE, NB))
